# baseline (speedup 1.0000x reference)
import functools

import jax
import jax.numpy as jnp
from jax import lax
from jax.experimental import pallas as pl
from jax.experimental.pallas import tpu as pltpu
from jax.experimental.pallas import tpu_sc as plsc

B = 4096
S = 200
D = 64
N = B * S
NC, NS = 2, 16
NW = NC * NS
ROWS_PER_W = N // NW
GROUP = 100
GROUPS_PER_CHUNK = 4
CHUNK = GROUP * GROUPS_PER_CHUNK
SEQ_PER_CHUNK = CHUNK // S
NCHUNKS = ROWS_PER_W // CHUNK
NBUF = 4
NITER = NCHUNKS // NBUF
NCHUNKS_TOTAL = N // CHUNK


def _emb_body(ids_hbm, table_hbm, pos_hbm, time_hbm, out_hbm, time_out,
              pos_v,
              ib0, ib1, ib2, ib3,
              rb0, rb1, rb2, rb3,
              gsem0, gsem1, gsem2, gsem3,
              ssem0, ssem1, ssem2, ssem3):
    ibs = [ib0, ib1, ib2, ib3]
    rbs = [rb0, rb1, rb2, rb3]
    gsems = [gsem0, gsem1, gsem2, gsem3]
    ssems = [ssem0, ssem1, ssem2, ssem3]

    wid = lax.axis_index("s") * NC + lax.axis_index("c")
    row0 = wid * ROWS_PER_W
    chunk0 = wid * NCHUNKS

    @pl.when(wid == 0)
    def _():
        pltpu.sync_copy(time_hbm, time_out)

    pltpu.sync_copy(pos_hbm.at[pl.ds(0, S)], pos_v)

    def fire(c, ib, rb, gsem):
        pltpu.sync_copy(ids_hbm.at[chunk0 + c], ib)
        for j in range(GROUPS_PER_CHUNK):
            pltpu.async_copy(table_hbm.at[ib.at[j]],
                             rb.at[pl.ds(j * GROUP, GROUP)], gsem)

    def drain_gathers(rb, gsem):
        pltpu.make_async_copy(out_hbm.at[pl.ds(0, CHUNK)], rb, gsem).wait()

    def wait_store(rb, ssem):
        pltpu.make_async_copy(rb, out_hbm.at[pl.ds(0, CHUNK)], ssem).wait()

    def add_pos(rb):
        def body(s, carry):
            p = [pos_v[s, pl.ds(16 * q, 16)] for q in range(4)]
            for t in range(SEQ_PER_CHUNK):
                r = t * S + s
                for q in range(4):
                    rb[r, pl.ds(16 * q, 16)] = rb[r, pl.ds(16 * q, 16)] + p[q]
            return carry
        lax.fori_loop(0, S, body, 0)

    for b in range(NBUF):
        fire(b, ibs[b], rbs[b], gsems[b])

    def outer(i, carry):
        for b in range(NBUF):
            c = i * NBUF + b
            drain_gathers(rbs[b], gsems[b])
            add_pos(rbs[b])
            pltpu.async_copy(rbs[b], out_hbm.at[pl.ds(row0 + c * CHUNK, CHUNK)],
                             ssems[b])
            pb = (b - 1) % NBUF
            cc = c + NBUF - 1
            @pl.when((c >= 1) & (cc < NCHUNKS))
            def _():
                wait_store(rbs[pb], ssems[pb])
                fire(cc, ibs[pb], rbs[pb], gsems[pb])
        return carry

    lax.fori_loop(0, NITER, outer, 0)

    for b in range(NBUF):
        wait_store(rbs[b], ssems[b])


@functools.partial(jax.jit, static_argnums=())
def _emb_lookup(ids2d, W_word, W_pos, W_time):
    mesh = plsc.VectorSubcoreMesh(core_axis_name="c", subcore_axis_name="s")
    kern = functools.partial(
        pl.kernel,
        mesh=mesh,
        compiler_params=pltpu.CompilerParams(use_tc_tiling_on_sc=False),
        out_type=[
            jax.ShapeDtypeStruct((N, D), jnp.float32),
            jax.ShapeDtypeStruct(W_time.shape, jnp.float32),
        ],
        scratch_types=(
            [pltpu.VMEM((S, D), jnp.float32)]
            + [pltpu.VMEM((GROUPS_PER_CHUNK, GROUP), jnp.int32) for _ in range(NBUF)]
            + [pltpu.VMEM((CHUNK, D), jnp.float32) for _ in range(NBUF)]
            + [pltpu.SemaphoreType.DMA for _ in range(2 * NBUF)]
        ),
    )(_emb_body)
    return kern(ids2d, W_word, W_pos, W_time)


def kernel(input_ids, W_word, W_pos, W_time):
    ids3d = input_ids.reshape(NCHUNKS_TOTAL, GROUPS_PER_CHUNK, GROUP)
    out_flat, time_emb = _emb_lookup(ids3d, W_word, W_pos, W_time)
    return out_flat.reshape(B, S, D), time_emb[None]
